# Initial kernel scaffold; baseline (speedup 1.0000x reference)
import jax
import jax.numpy as jnp
from jax.experimental import pallas as pl


def kernel(x, start, table):
    raise NotImplementedError("write your pallas kernel here")


# speedup vs baseline: 1.4948x; 1.4948x over previous
import functools

import jax
import jax.numpy as jnp
from jax.experimental import pallas as pl
from jax.experimental.pallas import tpu as pltpu

_BLK = 512


def _pe_add_body(start_ref, x_ref, pe_ref, o_ref):
    del start_ref
    o_ref[...] = x_ref[...] + pe_ref[...][None, :, :]


def kernel(x, start, table):
    B, L, D = x.shape
    blk = min(_BLK, L)
    num_j = L // blk
    start_arr = jnp.asarray(start, jnp.int32).reshape((1,))

    grid_spec = pltpu.PrefetchScalarGridSpec(
        num_scalar_prefetch=1,
        grid=(num_j, B),
        in_specs=[
            pl.BlockSpec((1, blk, D), lambda j, b, s: (b, j, 0)),
            pl.BlockSpec((blk, D), lambda j, b, s: (s[0] // blk + j, 0)),
        ],
        out_specs=pl.BlockSpec((1, blk, D), lambda j, b, s: (b, j, 0)),
    )
    return pl.pallas_call(
        _pe_add_body,
        grid_spec=grid_spec,
        out_shape=jax.ShapeDtypeStruct((B, L, D), x.dtype),
    )(start_arr, x, table)
